# baseline (speedup 1.0000x reference)
import functools

import jax
import jax.numpy as jnp
from jax.experimental import pallas as pl
from jax.experimental.pallas import tpu as pltpu

_T = 8
_GW = 256


def _lstm_body(x_ref, p_ref, wx_ref, wh_ref, b_ref, h_out, c_out,
               xp_ref, h_ref, c_ref, *, T, B, U):
    it = pl.program_id(0)
    ng = U // _GW

    @pl.when(it == 0)
    def _init():
        h_ref[...] = jnp.zeros_like(h_ref)
        c_ref[...] = jnp.zeros_like(c_ref)

    xs_b = x_ref[...].astype(jnp.bfloat16).reshape(B * T, x_ref.shape[2])
    xs_t = jnp.dot(p_ref[...], xs_b,
                   preferred_element_type=jnp.float32).astype(jnp.bfloat16)
    xp_ref[...] = (
        jnp.dot(xs_t, wx_ref[...], preferred_element_type=jnp.float32)
        + b_ref[...]
    )

    h_val = h_ref[...]
    for t in range(T):
        gates = xp_ref[pl.ds(t * B, B), :] + jnp.dot(
            h_val, wh_ref[...], preferred_element_type=jnp.float32)
        h_chunks = []
        for n in range(ng):
            blk = gates[:, n * 4 * _GW:(n + 1) * 4 * _GW]
            ti = jnp.tanh(blk[:, 0 * _GW:1 * _GW])
            tf = jnp.tanh(blk[:, 1 * _GW:2 * _GW])
            g = jnp.tanh(blk[:, 2 * _GW:3 * _GW])
            to = jnp.tanh(blk[:, 3 * _GW:4 * _GW])
            csl = pl.ds(n * _GW, _GW)
            c_n = ((0.5 * tf + 0.5) * c_ref[:, csl]
                   + (0.5 * ti + 0.5) * g)
            c_ref[:, csl] = c_n
            hn = (0.5 * to + 0.5) * jnp.tanh(c_n)
            h_chunks.append(hn.astype(jnp.bfloat16))
            if t == T - 1:
                h_out[:, csl] = hn
                c_out[:, csl] = c_n
        h_val = jnp.concatenate(h_chunks, axis=1)

    h_ref[...] = h_val


@jax.jit
def kernel(x, Wx, Wh, b):
    B, S, D = x.shape
    U = Wh.shape[0]
    G = 4 * U
    T = _T

    cols = jnp.arange(G)
    n = cols // (4 * _GW)
    q = (cols % (4 * _GW)) // _GW
    off = cols % _GW
    src = q * U + n * _GW + off
    scale = jnp.where(q == 2, 1.0, 0.5)
    wx = (Wx[:, src] * scale).astype(jnp.bfloat16)
    wh = (Wh[:, src] * scale).astype(jnp.bfloat16)
    b2 = (b[src] * scale).astype(jnp.float32).reshape(1, G)
    rows = jnp.arange(T * B)
    rsrc = (rows % B) * T + rows // B
    perm = (rsrc[:, None] == jnp.arange(B * T)[None, :]).astype(jnp.bfloat16)

    body = functools.partial(_lstm_body, T=T, B=B, U=U)
    h, c = pl.pallas_call(
        body,
        out_shape=[
            jax.ShapeDtypeStruct((B, U), jnp.float32),
            jax.ShapeDtypeStruct((B, U), jnp.float32),
        ],
        grid=(S // T,),
        in_specs=[
            pl.BlockSpec((B, T, D), lambda it: (0, it, 0)),
            pl.BlockSpec((T * B, T * B), lambda it: (0, 0)),
            pl.BlockSpec((D, G), lambda it: (0, 0)),
            pl.BlockSpec((U, G), lambda it: (0, 0)),
            pl.BlockSpec((1, G), lambda it: (0, 0)),
        ],
        out_specs=[
            pl.BlockSpec((B, U), lambda it: (0, 0)),
            pl.BlockSpec((B, U), lambda it: (0, 0)),
        ],
        scratch_shapes=[
            pltpu.VMEM((T * B, G), jnp.float32),
            pltpu.VMEM((B, U), jnp.bfloat16),
            pltpu.VMEM((B, U), jnp.float32),
        ],
        compiler_params=pltpu.CompilerParams(
            dimension_semantics=("arbitrary",),
            vmem_limit_bytes=56 * 1024 * 1024,
        ),
        name="lstm_fused",
    )(x, perm, wx, wh, b2)
    return h, c
